# baseline (speedup 1.0000x reference)
import functools

import jax
import jax.numpy as jnp
from jax import lax
from jax.experimental import pallas as pl
from jax.experimental.pallas import tpu as pltpu
from jax.experimental.pallas import tpu_sc as plsc

VOCAB = 1000
EMB = 64
HID = 128
B = 1024
T = 50
BT = B * T

NC = 2
NS = 16
NW = NC * NS
EMB_P = 128
PER_W = BT // NW
CH = 80
NPASS = 2
HALF = PER_W // NPASS
NCH = HALF // CH


@functools.lru_cache(maxsize=1)
def _make_sc_gather():
    mesh = plsc.VectorSubcoreMesh(
        core_axis_name="c", subcore_axis_name="s", num_cores=NC, num_subcores=NS
    )

    @functools.partial(
        pl.kernel,
        out_type=jax.ShapeDtypeStruct((BT, EMB_P), jnp.float32),
        mesh=mesh,
        scratch_types=[
            pltpu.VMEM((PER_W,), jnp.int32),
            pltpu.VMEM((HALF, EMB_P), jnp.float32),
            pltpu.SemaphoreType.DMA,
        ],
    )
    def _sc_gather(idx_hbm, emb_hbm, out_hbm, idx_v, rows_v, sem):
        wid = lax.axis_index("s") * NC + lax.axis_index("c")
        base = wid * PER_W
        pltpu.sync_copy(idx_hbm.at[pl.ds(base, PER_W)], idx_v)

        for p in range(NPASS):
            def fire(j, carry, p=p):
                pltpu.async_copy(
                    emb_hbm.at[idx_v.at[pl.ds(p * HALF + j * CH, CH)]],
                    rows_v.at[pl.ds(j * CH, CH)],
                    sem,
                )
                return carry

            lax.fori_loop(0, NCH, fire, 0)

            def drain(j, carry, p=p):
                pltpu.make_async_copy(
                    emb_hbm.at[idx_v.at[pl.ds(p * HALF + j * CH, CH)]],
                    rows_v.at[pl.ds(j * CH, CH)],
                    sem,
                ).wait()
                return carry

            lax.fori_loop(0, NCH, drain, 0)
            pltpu.sync_copy(rows_v, out_hbm.at[pl.ds(base + p * HALF, HALF)])

    return _sc_gather


def _gather_embeds(idx, emb_padded):
    return _make_sc_gather()(idx, emb_padded)


BBLK = 256
TCH = 8
NB = B // BBLK
NT = (T + TCH - 1) // TCH
BF = jnp.bfloat16
NS_P = 4
PCOL = BBLK * T // NS_P


def _lstm_body(ex_ref, wih_ref, whh_ref, bih_ref, bhh_ref, wlin_ref, blin_ref,
               out_hbm, hn_ref, cn_ref, h_ref, c_ref, hs_ref, res_ref, sems):
    bb = pl.program_id(0)
    tj = pl.program_id(1)

    @pl.when(tj == 0)
    def _():
        h_ref[...] = jnp.zeros_like(h_ref)
        c_ref[...] = jnp.zeros_like(c_ref)

    b = bih_ref[...] + bhh_ref[...]
    ex2 = ex_ref[...].reshape(TCH * BBLK, EMB_P).astype(BF)
    gx = jnp.dot(ex2, wih_ref[...], preferred_element_type=jnp.float32) + b
    gx3 = gx.reshape(TCH, BBLK, 4 * HID)

    whh = whh_ref[...]
    h = h_ref[...]
    c = c_ref[...]
    for t in range(TCH):
        gt = tj * TCH + t
        valid = gt < T
        gates = gx3[t] + jnp.dot(h.astype(BF), whh,
                                 preferred_element_type=jnp.float32)
        i = 0.5 * jnp.tanh(0.5 * gates[:, :HID]) + 0.5
        f = 0.5 * jnp.tanh(0.5 * gates[:, HID:2 * HID]) + 0.5
        g = jnp.tanh(gates[:, 2 * HID:3 * HID])
        o = 0.5 * jnp.tanh(0.5 * gates[:, 3 * HID:]) + 0.5
        c = jnp.where(valid, f * c + i * g, c)
        h = jnp.where(valid, o * jnp.tanh(c), h)

        @pl.when(valid)
        def _(t=t, gt=gt, h=h):
            hs_ref[:, tj * TCH + t, :] = h
    h_ref[...] = h
    c_ref[...] = c
    hn_ref[...] = h
    cn_ref[...] = c

    @pl.when(tj == NT - 1)
    def _():
        hs2 = hs_ref[...].reshape(BBLK * T, HID).astype(BF)
        blin = blin_ref[...]
        wlin = wlin_ref[...]
        for s in range(NS_P):
            m = bb * NS_P + s
            slot = lax.rem(m, 2)

            @pl.when(m >= 2)
            def _(slot=slot):
                pltpu.make_async_copy(
                    res_ref.at[slot],
                    out_hbm.at[:, pl.ds(0, PCOL)],
                    sems.at[slot],
                ).wait()

            rq = lax.dot_general(
                wlin, hs2[s * PCOL:(s + 1) * PCOL, :],
                (((1,), (1,)), ((), ())),
                preferred_element_type=jnp.float32,
            )
            res_ref[slot] = rq + blin
            pltpu.async_copy(
                res_ref.at[slot],
                out_hbm.at[:, pl.ds(m * PCOL, PCOL)],
                sems.at[slot],
            )

        @pl.when(bb == NB - 1)
        def _():
            for slot in range(2):
                pltpu.make_async_copy(
                    res_ref.at[slot],
                    out_hbm.at[:, pl.ds(0, PCOL)],
                    sems.at[slot],
                ).wait()


def _lstm_call(embeds_tm, wihT, whhT, bih2, bhh2, wlin_bf, blin_col):
    return pl.pallas_call(
        _lstm_body,
        grid=(NB, NT),
        in_specs=[
            pl.BlockSpec((TCH, BBLK, EMB_P), lambda i, j: (j, i, 0)),
            pl.BlockSpec((EMB_P, 4 * HID), lambda i, j: (0, 0)),
            pl.BlockSpec((HID, 4 * HID), lambda i, j: (0, 0)),
            pl.BlockSpec((1, 4 * HID), lambda i, j: (0, 0)),
            pl.BlockSpec((1, 4 * HID), lambda i, j: (0, 0)),
            pl.BlockSpec((VOCAB, HID), lambda i, j: (0, 0)),
            pl.BlockSpec((VOCAB, 1), lambda i, j: (0, 0)),
        ],
        out_specs=[
            pl.BlockSpec(memory_space=pltpu.HBM),
            pl.BlockSpec((BBLK, HID), lambda i, j: (i, 0)),
            pl.BlockSpec((BBLK, HID), lambda i, j: (i, 0)),
        ],
        out_shape=[
            jax.ShapeDtypeStruct((VOCAB, BT), jnp.float32),
            jax.ShapeDtypeStruct((B, HID), jnp.float32),
            jax.ShapeDtypeStruct((B, HID), jnp.float32),
        ],
        scratch_shapes=[
            pltpu.VMEM((BBLK, HID), jnp.float32),
            pltpu.VMEM((BBLK, HID), jnp.float32),
            pltpu.VMEM((BBLK, T, HID), jnp.float32),
            pltpu.VMEM((2, VOCAB, PCOL), jnp.float32),
            pltpu.SemaphoreType.DMA((2,)),
        ],
        compiler_params=pltpu.CompilerParams(
            dimension_semantics=("arbitrary", "arbitrary"),
        ),
    )(embeds_tm, wihT, whhT, bih2, bhh2, wlin_bf, blin_col)


def kernel(x, emb, w_ih, w_hh, b_ih, b_hh, w_lin, b_lin):
    idx = x.astype(jnp.int32).T.reshape(BT)
    emb_padded = jnp.pad(emb, ((0, 0), (0, EMB_P - EMB)))
    embeds = _gather_embeds(idx, emb_padded)
    embeds_tm = embeds.reshape(T, B, EMB_P)
    wihT = jnp.pad(w_ih.T, ((0, EMB_P - EMB), (0, 0))).astype(BF)
    logits_t, hn, cn = _lstm_call(
        embeds_tm,
        wihT,
        w_hh.T.astype(BF),
        b_ih.reshape(1, 4 * HID),
        b_hh.reshape(1, 4 * HID),
        w_lin.astype(BF),
        b_lin.reshape(VOCAB, 1),
    )
    return (logits_t.T, hn[None], cn[None])


# speedup vs baseline: 1.2059x; 1.2059x over previous
import functools

import jax
import jax.numpy as jnp
from jax import lax
from jax.experimental import pallas as pl
from jax.experimental.pallas import tpu as pltpu
from jax.experimental.pallas import tpu_sc as plsc

VOCAB = 1000
EMB = 64
HID = 128
B = 1024
T = 50
BT = B * T

NC = 2
NS = 16
NW = NC * NS
EMB_P = 128
PER_W = BT // NW
CH = 80
NPASS = 2
HALF = PER_W // NPASS
NCH = HALF // CH


@functools.lru_cache(maxsize=1)
def _make_sc_gather():
    mesh = plsc.VectorSubcoreMesh(
        core_axis_name="c", subcore_axis_name="s", num_cores=NC, num_subcores=NS
    )

    @functools.partial(
        pl.kernel,
        out_type=jax.ShapeDtypeStruct((BT, EMB_P), jnp.float32),
        mesh=mesh,
        scratch_types=[
            pltpu.VMEM((PER_W,), jnp.int32),
            pltpu.VMEM((HALF, EMB_P), jnp.float32),
            pltpu.SemaphoreType.DMA,
        ],
    )
    def _sc_gather(idx_hbm, emb_hbm, out_hbm, idx_v, rows_v, sem):
        wid = lax.axis_index("s") * NC + lax.axis_index("c")
        base = wid * PER_W
        pltpu.sync_copy(idx_hbm.at[pl.ds(base, PER_W)], idx_v)

        for p in range(NPASS):
            def fire(j, carry, p=p):
                pltpu.async_copy(
                    emb_hbm.at[idx_v.at[pl.ds(p * HALF + j * CH, CH)]],
                    rows_v.at[pl.ds(j * CH, CH)],
                    sem,
                )
                return carry

            lax.fori_loop(0, NCH, fire, 0)

            def drain(j, carry, p=p):
                pltpu.make_async_copy(
                    emb_hbm.at[idx_v.at[pl.ds(p * HALF + j * CH, CH)]],
                    rows_v.at[pl.ds(j * CH, CH)],
                    sem,
                ).wait()
                return carry

            lax.fori_loop(0, NCH, drain, 0)
            pltpu.sync_copy(rows_v, out_hbm.at[pl.ds(base + p * HALF, HALF)])

    return _sc_gather


def _gather_embeds(idx, emb_padded):
    return _make_sc_gather()(idx, emb_padded)


BBLK = 1024
TCH = 8
NB = B // BBLK
NT = (T + TCH - 1) // TCH
TP = NT * TCH
BF = jnp.bfloat16


def _lstm_body(ex_ref, wih_ref, whh_ref, bih_ref, bhh_ref,
               hs_out, hn_ref, cn_ref, h_ref, c_ref):
    tj = pl.program_id(1)

    @pl.when(tj == 0)
    def _():
        h_ref[...] = jnp.zeros_like(h_ref)
        c_ref[...] = jnp.zeros_like(c_ref)

    b = bih_ref[...] + bhh_ref[...]
    ex2 = ex_ref[...].reshape(TCH * BBLK, EMB_P).astype(BF)
    gx = jnp.dot(ex2, wih_ref[...], preferred_element_type=jnp.float32) + b
    gx3 = gx.reshape(TCH, BBLK, 4 * HID)

    whh = whh_ref[...]
    h = h_ref[...]
    c = c_ref[...]
    for t in range(TCH):
        valid = tj * TCH + t < T
        gates = gx3[t] + jnp.dot(h.astype(BF), whh,
                                 preferred_element_type=jnp.float32)
        i = 0.5 * jnp.tanh(0.5 * gates[:, :HID]) + 0.5
        f = 0.5 * jnp.tanh(0.5 * gates[:, HID:2 * HID]) + 0.5
        g = jnp.tanh(gates[:, 2 * HID:3 * HID])
        o = 0.5 * jnp.tanh(0.5 * gates[:, 3 * HID:]) + 0.5
        c = jnp.where(valid, f * c + i * g, c)
        h = jnp.where(valid, o * jnp.tanh(c), h)
        hs_out[:, t, :] = h
    h_ref[...] = h
    c_ref[...] = c
    hn_ref[...] = h
    cn_ref[...] = c


def _lstm_call(embeds_tm, wihT, whhT, bih2, bhh2):
    return pl.pallas_call(
        _lstm_body,
        grid=(NB, NT),
        in_specs=[
            pl.BlockSpec((TCH, BBLK, EMB_P), lambda i, j: (j, i, 0)),
            pl.BlockSpec((EMB_P, 4 * HID), lambda i, j: (0, 0)),
            pl.BlockSpec((HID, 4 * HID), lambda i, j: (0, 0)),
            pl.BlockSpec((1, 4 * HID), lambda i, j: (0, 0)),
            pl.BlockSpec((1, 4 * HID), lambda i, j: (0, 0)),
        ],
        out_specs=[
            pl.BlockSpec((BBLK, TCH, HID), lambda i, j: (i, j, 0)),
            pl.BlockSpec((BBLK, HID), lambda i, j: (i, 0)),
            pl.BlockSpec((BBLK, HID), lambda i, j: (i, 0)),
        ],
        out_shape=[
            jax.ShapeDtypeStruct((B, T, HID), jnp.float32),
            jax.ShapeDtypeStruct((B, HID), jnp.float32),
            jax.ShapeDtypeStruct((B, HID), jnp.float32),
        ],
        scratch_shapes=[
            pltpu.VMEM((BBLK, HID), jnp.float32),
            pltpu.VMEM((BBLK, HID), jnp.float32),
        ],
        compiler_params=pltpu.CompilerParams(
            dimension_semantics=("arbitrary", "arbitrary"),
        ),
    )(embeds_tm, wihT, whhT, bih2, bhh2)


PB = 64
PG = PB // 4
NPB = B // PB


def _proj_body(hs_ref, wlin_ref, blin_ref, out_ref):
    hs2 = hs_ref[...].reshape(PB * T, HID).astype(BF)
    rqT = lax.dot_general(wlin_ref[...], hs2, (((1,), (1,)), ((), ())),
                          preferred_element_type=jnp.float32)
    out_ref[...] = rqT + blin_ref[...]


def _proj_call(hs3, wlin_bf, blin_col):
    return pl.pallas_call(
        _proj_body,
        grid=(NPB,),
        in_specs=[
            pl.BlockSpec((PB, T, HID), lambda i: (i, 0, 0)),
            pl.BlockSpec((VOCAB, HID), lambda i: (0, 0)),
            pl.BlockSpec((VOCAB, 1), lambda i: (0, 0)),
        ],
        out_specs=pl.BlockSpec((VOCAB, PB * T), lambda i: (0, i)),
        out_shape=jax.ShapeDtypeStruct((VOCAB, BT), jnp.float32),
        compiler_params=pltpu.CompilerParams(
            dimension_semantics=("arbitrary",),
        ),
    )(hs3, wlin_bf, blin_col)


def kernel(x, emb, w_ih, w_hh, b_ih, b_hh, w_lin, b_lin):
    idx = x.astype(jnp.int32).T.reshape(BT)
    emb_padded = jnp.pad(emb, ((0, 0), (0, EMB_P - EMB)))
    embeds = _gather_embeds(idx, emb_padded)
    embeds_tm = embeds.reshape(T, B, EMB_P)
    wihT = jnp.pad(w_ih.T, ((0, EMB_P - EMB), (0, 0))).astype(BF)
    hs3, hn, cn = _lstm_call(
        embeds_tm,
        wihT,
        w_hh.T.astype(BF),
        b_ih.reshape(1, 4 * HID),
        b_hh.reshape(1, 4 * HID),
    )
    logits_t = _proj_call(hs3, w_lin.astype(BF), b_lin.reshape(VOCAB, 1))
    return (logits_t.T, hn[None], cn[None])
